# baseline (speedup 1.0000x reference)
import jax
import jax.numpy as jnp
import numpy as np
from jax import lax
from jax.experimental import pallas as pl
from jax.experimental.pallas import tpu as pltpu
from jax.experimental.pallas import tpu_sc as plsc

AA = 20
MP = 15
F = 3
NP = 34
LOUT = AA + F - 1
XC = 1 + MP * AA + NP
LANES = 16
NWORK = 32
BLK = 128
NCH = BLK // LANES
XA_ROWS = 184
XP_OFF = 296
XP_ROWS = 39
GROUPS = ((0, 12), (12, 12), (24, 10))
GMAX = 12

_P2J = {0: [0], 1: [1, 2], 2: [0, 1], 3: [2], 4: [1], 6: [2, 3], 7: [3],
        10: [4], 12: [5], 14: [6, 7], 15: [7], 17: [8], 18: [5, 6], 19: [7],
        21: [8], 22: [7, 8], 24: [8], 25: [6], 27: [4], 28: [3], 30: [2],
        31: [1], 33: [0]}


def _body(xh, wh, oh, wv, xa0, xa1, xp, ovA, ovB, si0, si1, soA, soB, nblk):
    cid = lax.axis_index("c")
    sid = lax.axis_index("s")
    wid = sid * 2 + cid
    npair = nblk // 2

    pltpu.sync_copy(wh, wv)

    zero = jnp.zeros((LANES,), jnp.float32)

    def hbase(i):
        return (wid * nblk + i) * BLK

    def xa_slice(i):
        return xh.at[pl.ds(0, XA_ROWS), pl.ds(hbase(i), BLK)]

    def start_in(i, buf, sem):
        pltpu.async_copy(xa_slice(i), buf, sem)

    def wait_in(i, buf, sem):
        pltpu.make_async_copy(xa_slice(i), buf, sem).wait()

    def out_pair(i, g, ov):
        p0, n = GROUPS[g]
        return (ov.at[pl.ds(0, n), pl.ds(0, LOUT), :],
                oh.at[pl.ds(p0, n), :, pl.ds(hbase(i), BLK)])

    def start_out(i, g, ov, sem):
        src, dst = out_pair(i, g, ov)
        pltpu.async_copy(src, dst, sem)

    def wait_out(i, g, ov, sem):
        src, dst = out_pair(i, g, ov)
        pltpu.make_async_copy(src, dst, sem).wait()

    def group_compute(g, ov, xa):
        p0, n = GROUPS[g]

        @plsc.parallel_loop(0, NCH)
        def chunk(c):
            off = c * LANES
            pep = {}

            def pvec(j, a):
                if (j, a) not in pep:
                    pep[(j, a)] = xa[1 + j * AA + a, pl.ds(off, LANES)]
                return pep[(j, a)]

            aggs = {}
            for p in range(p0, p0 + n):
                js = _P2J.get(p)
                if js is None or tuple(js) in aggs:
                    continue
                if len(js) == 1:
                    aggs[tuple(js)] = [pvec(js[0], a) for a in range(AA)]
                else:
                    aggs[tuple(js)] = [pvec(js[0], a) + pvec(js[1], a)
                                       for a in range(AA)]
            for p in range(p0, p0 + n):
                js = _P2J.get(p)
                if js is None:
                    for l in range(LOUT):
                        ov[p - p0, l, pl.ds(off, LANES)] = zero
                    continue
                agg = aggs[tuple(js)]
                fb = (xp[301 - XP_OFF + p, pl.ds(off, LANES)]
                      .astype(jnp.int32) * F)
                f = [plsc.load_gather(wv, [fb + t]) for t in range(F)]
                s = [[f[t] * agg[a] for a in range(AA)] for t in range(F)]
                for l in range(LOUT):
                    acc = None
                    for t in range(F):
                        a = l - t
                        if 0 <= a < AA:
                            acc = s[t][a] if acc is None else acc + s[t][a]
                    ov[p - p0, l, pl.ds(off, LANES)] = acc

    start_in(0, xa0, si0)

    def pair(j, carry):
        e = 2 * j
        o = e + 1
        start_in(o, xa1, si1)
        pltpu.sync_copy(xh.at[pl.ds(XP_OFF, XP_ROWS), pl.ds(hbase(e), BLK)],
                        xp)
        wait_in(e, xa0, si0)

        @pl.when(j > 0)
        def _():
            wait_out(e - 1, 1, ovA, soA)
            wait_out(e - 1, 2, ovB, soB)

        group_compute(0, ovA, xa0)
        start_out(e, 0, ovA, soA)
        group_compute(1, ovB, xa0)
        start_out(e, 1, ovB, soB)
        wait_out(e, 0, ovA, soA)
        group_compute(2, ovA, xa0)
        start_out(e, 2, ovA, soA)
        @pl.when(j + 1 < npair)
        def _():
            start_in(e + 2, xa0, si0)
        pltpu.sync_copy(xh.at[pl.ds(XP_OFF, XP_ROWS), pl.ds(hbase(o), BLK)],
                        xp)
        wait_in(o, xa1, si1)
        wait_out(e, 1, ovB, soB)
        group_compute(0, ovB, xa1)
        start_out(o, 0, ovB, soB)
        wait_out(e, 2, ovA, soA)
        group_compute(1, ovA, xa1)
        start_out(o, 1, ovA, soA)
        wait_out(o, 0, ovB, soB)
        group_compute(2, ovB, xa1)
        start_out(o, 2, ovB, soB)
        return carry

    lax.fori_loop(0, npair, pair, 0)
    wait_out(nblk - 1, 1, ovA, soA)
    wait_out(nblk - 1, 2, ovB, soB)


def kernel(x, W):
    B = x.shape[0]
    nblk = B // (NWORK * BLK)
    mesh = plsc.VectorSubcoreMesh(core_axis_name="c", subcore_axis_name="s")
    run = pl.kernel(
        lambda xh, wh, oh, wv, xa0, xa1, xp, ovA, ovB, si0, si1, soA, soB:
            _body(xh, wh, oh, wv, xa0, xa1, xp, ovA, ovB,
                  si0, si1, soA, soB, nblk),
        out_type=jax.ShapeDtypeStruct((NP, LOUT, B), jnp.float32),
        mesh=mesh,
        scratch_types=[
            pltpu.VMEM((AA * F,), jnp.float32),
            pltpu.VMEM((XA_ROWS, BLK), jnp.float32),
            pltpu.VMEM((XA_ROWS, BLK), jnp.float32),
            pltpu.VMEM((XP_ROWS, BLK), jnp.float32),
            pltpu.VMEM((GMAX, LOUT + 2, BLK), jnp.float32),
            pltpu.VMEM((GMAX, LOUT + 2, BLK), jnp.float32),
            pltpu.SemaphoreType.DMA,
            pltpu.SemaphoreType.DMA,
            pltpu.SemaphoreType.DMA,
            pltpu.SemaphoreType.DMA,
        ],
        compiler_params=pltpu.CompilerParams(needs_layout_passes=False,
                                             use_tc_tiling_on_sc=True),
    )
    out = run(x.T, W.reshape(-1))
    return jnp.transpose(out, (2, 0, 1))


# speedup vs baseline: 1.0006x; 1.0006x over previous
import jax
import jax.numpy as jnp
import numpy as np
from jax import lax
from jax.experimental import pallas as pl
from jax.experimental.pallas import tpu as pltpu
from jax.experimental.pallas import tpu_sc as plsc

AA = 20
MP = 15
F = 3
NP = 34
LOUT = AA + F - 1
XC = 1 + MP * AA + NP
LANES = 16
NWORK = 32
BLK = 128
NCH = BLK // LANES
XA_ROWS = 184
XP_OFF = 296
XP_ROWS = 39
GROUPS = ((0, 12), (12, 12), (24, 10))
GMAX = 12

_P2J = {0: [0], 1: [1, 2], 2: [0, 1], 3: [2], 4: [1], 6: [2, 3], 7: [3],
        10: [4], 12: [5], 14: [6, 7], 15: [7], 17: [8], 18: [5, 6], 19: [7],
        21: [8], 22: [7, 8], 24: [8], 25: [6], 27: [4], 28: [3], 30: [2],
        31: [1], 33: [0]}


def _body(xh, wh, oh, wv, xa0, xa1, xp, ovA, ovB, si0, si1, soA, soB, nblk):
    cid = lax.axis_index("c")
    sid = lax.axis_index("s")
    wid = sid * 2 + cid
    npair = nblk // 2

    pltpu.sync_copy(wh, wv)

    zero = jnp.zeros((LANES,), jnp.float32)

    def hbase(i):
        return (wid * nblk + i) * BLK

    def xa_slice(i):
        return xh.at[pl.ds(0, XA_ROWS), pl.ds(hbase(i), BLK)]

    def start_in(i, buf, sem):
        pltpu.async_copy(xa_slice(i), buf, sem)

    def wait_in(i, buf, sem):
        pltpu.make_async_copy(xa_slice(i), buf, sem).wait()

    def out_pair(i, g, ov):
        p0, n = GROUPS[g]
        return (ov.at[pl.ds(0, n), pl.ds(0, LOUT), :],
                oh.at[pl.ds(p0, n), :, pl.ds(hbase(i), BLK)])

    def start_out(i, g, ov, sem):
        src, dst = out_pair(i, g, ov)
        pltpu.async_copy(src, dst, sem)

    def wait_out(i, g, ov, sem):
        src, dst = out_pair(i, g, ov)
        pltpu.make_async_copy(src, dst, sem).wait()

    def group_compute(g, ov, xa):
        p0, n = GROUPS[g]

        def chunk(c, carry):
            off = c * LANES
            pep = {}

            def pvec(j, a):
                if (j, a) not in pep:
                    pep[(j, a)] = xa[1 + j * AA + a, pl.ds(off, LANES)]
                return pep[(j, a)]

            aggs = {}
            for p in range(p0, p0 + n):
                js = _P2J.get(p)
                if js is None or tuple(js) in aggs:
                    continue
                if len(js) == 1:
                    aggs[tuple(js)] = [pvec(js[0], a) for a in range(AA)]
                else:
                    aggs[tuple(js)] = [pvec(js[0], a) + pvec(js[1], a)
                                       for a in range(AA)]
            for p in range(p0, p0 + n):
                js = _P2J.get(p)
                if js is None:
                    for l in range(LOUT):
                        ov[p - p0, l, pl.ds(off, LANES)] = zero
                    continue
                agg = aggs[tuple(js)]
                fb = (xp[301 - XP_OFF + p, pl.ds(off, LANES)]
                      .astype(jnp.int32) * F)
                f = [plsc.load_gather(wv, [fb + t]) for t in range(F)]
                s = [[f[t] * agg[a] for a in range(AA)] for t in range(F)]
                for l in range(LOUT):
                    acc = None
                    for t in range(F):
                        a = l - t
                        if 0 <= a < AA:
                            acc = s[t][a] if acc is None else acc + s[t][a]
                    ov[p - p0, l, pl.ds(off, LANES)] = acc
            return carry

        lax.fori_loop(0, NCH, chunk, 0)

    start_in(0, xa0, si0)

    def pair(j, carry):
        e = 2 * j
        o = e + 1
        start_in(o, xa1, si1)
        pltpu.sync_copy(xh.at[pl.ds(XP_OFF, XP_ROWS), pl.ds(hbase(e), BLK)],
                        xp)
        wait_in(e, xa0, si0)

        @pl.when(j > 0)
        def _():
            wait_out(e - 1, 1, ovA, soA)
            wait_out(e - 1, 2, ovB, soB)

        group_compute(0, ovA, xa0)
        start_out(e, 0, ovA, soA)
        group_compute(1, ovB, xa0)
        start_out(e, 1, ovB, soB)
        wait_out(e, 0, ovA, soA)
        group_compute(2, ovA, xa0)
        start_out(e, 2, ovA, soA)
        @pl.when(j + 1 < npair)
        def _():
            start_in(e + 2, xa0, si0)
        pltpu.sync_copy(xh.at[pl.ds(XP_OFF, XP_ROWS), pl.ds(hbase(o), BLK)],
                        xp)
        wait_in(o, xa1, si1)
        wait_out(e, 1, ovB, soB)
        group_compute(0, ovB, xa1)
        start_out(o, 0, ovB, soB)
        wait_out(e, 2, ovA, soA)
        group_compute(1, ovA, xa1)
        start_out(o, 1, ovA, soA)
        wait_out(o, 0, ovB, soB)
        group_compute(2, ovB, xa1)
        start_out(o, 2, ovB, soB)
        return carry

    lax.fori_loop(0, npair, pair, 0)
    wait_out(nblk - 1, 1, ovA, soA)
    wait_out(nblk - 1, 2, ovB, soB)


def kernel(x, W):
    B = x.shape[0]
    nblk = B // (NWORK * BLK)
    mesh = plsc.VectorSubcoreMesh(core_axis_name="c", subcore_axis_name="s")
    run = pl.kernel(
        lambda xh, wh, oh, wv, xa0, xa1, xp, ovA, ovB, si0, si1, soA, soB:
            _body(xh, wh, oh, wv, xa0, xa1, xp, ovA, ovB,
                  si0, si1, soA, soB, nblk),
        out_type=jax.ShapeDtypeStruct((NP, LOUT, B), jnp.float32),
        mesh=mesh,
        scratch_types=[
            pltpu.VMEM((AA * F,), jnp.float32),
            pltpu.VMEM((XA_ROWS, BLK), jnp.float32),
            pltpu.VMEM((XA_ROWS, BLK), jnp.float32),
            pltpu.VMEM((XP_ROWS, BLK), jnp.float32),
            pltpu.VMEM((GMAX, LOUT + 2, BLK), jnp.float32),
            pltpu.VMEM((GMAX, LOUT + 2, BLK), jnp.float32),
            pltpu.SemaphoreType.DMA,
            pltpu.SemaphoreType.DMA,
            pltpu.SemaphoreType.DMA,
            pltpu.SemaphoreType.DMA,
        ],
        compiler_params=pltpu.CompilerParams(needs_layout_passes=False,
                                             use_tc_tiling_on_sc=True),
    )
    out = run(x.T, W.reshape(-1))
    return jnp.transpose(out, (2, 0, 1))


# speedup vs baseline: 1.1271x; 1.1264x over previous
import jax
import jax.numpy as jnp
import numpy as np
from jax import lax
from jax.experimental import pallas as pl
from jax.experimental.pallas import tpu as pltpu
from jax.experimental.pallas import tpu_sc as plsc

AA = 20
MP = 15
F = 3
NP = 34
LOUT = AA + F - 1
XC = 1 + MP * AA + NP
LANES = 16
NWORK = 32
BLK = 128
NCH = BLK // LANES
XA_ROWS = 184
XP_OFF = 296
XP_ROWS = 39
GROUPS = ((0, 9), (9, 9), (18, 8), (26, 8))
GMAX = 9

_P2J = {0: [0], 1: [1, 2], 2: [0, 1], 3: [2], 4: [1], 6: [2, 3], 7: [3],
        10: [4], 12: [5], 14: [6, 7], 15: [7], 17: [8], 18: [5, 6], 19: [7],
        21: [8], 22: [7, 8], 24: [8], 25: [6], 27: [4], 28: [3], 30: [2],
        31: [1], 33: [0]}


def _body(xh, wh, oh, wv, xa0, xa1, xp0, xp1, ovA, ovB,
          si0, si1, soA, soB, nblk):
    cid = lax.axis_index("c")
    sid = lax.axis_index("s")
    wid = sid * 2 + cid
    npair = nblk // 2

    pltpu.sync_copy(wh, wv)

    zero = jnp.zeros((LANES,), jnp.float32)

    def hbase(i):
        return (wid * nblk + i) * BLK

    def in_pairs(i, xa, xp):
        return ((xh.at[pl.ds(0, XA_ROWS), pl.ds(hbase(i), BLK)], xa),
                (xh.at[pl.ds(XP_OFF, XP_ROWS), pl.ds(hbase(i), BLK)], xp))

    def start_in(i, xa, xp, sem):
        for src, dst in in_pairs(i, xa, xp):
            pltpu.async_copy(src, dst, sem)

    def wait_in(i, xa, xp, sem):
        for src, dst in in_pairs(i, xa, xp):
            pltpu.make_async_copy(src, dst, sem).wait()

    def out_pair(i, g, ov):
        p0, n = GROUPS[g]
        return (ov.at[pl.ds(0, n), pl.ds(0, LOUT), :],
                oh.at[pl.ds(p0, n), :, pl.ds(hbase(i), BLK)])

    def start_out(i, g, ov, sem):
        src, dst = out_pair(i, g, ov)
        pltpu.async_copy(src, dst, sem)

    def wait_out(i, g, ov, sem):
        src, dst = out_pair(i, g, ov)
        pltpu.make_async_copy(src, dst, sem).wait()

    def group_compute(g, ov, xa, xp):
        p0, n = GROUPS[g]

        def chunk(c, carry):
            off = c * LANES
            pep = {}

            def pvec(j, a):
                if (j, a) not in pep:
                    pep[(j, a)] = xa[1 + j * AA + a, pl.ds(off, LANES)]
                return pep[(j, a)]

            aggs = {}
            for p in range(p0, p0 + n):
                js = _P2J.get(p)
                if js is None or tuple(js) in aggs:
                    continue
                if len(js) == 1:
                    aggs[tuple(js)] = [pvec(js[0], a) for a in range(AA)]
                else:
                    aggs[tuple(js)] = [pvec(js[0], a) + pvec(js[1], a)
                                       for a in range(AA)]
            for p in range(p0, p0 + n):
                js = _P2J.get(p)
                if js is None:
                    for l in range(LOUT):
                        ov[p - p0, l, pl.ds(off, LANES)] = zero
                    continue
                agg = aggs[tuple(js)]
                fb = (xp[301 - XP_OFF + p, pl.ds(off, LANES)]
                      .astype(jnp.int32) * F)
                f = [plsc.load_gather(wv, [fb + t]) for t in range(F)]
                for l in range(LOUT):
                    acc = None
                    for t in range(F):
                        a = l - t
                        if 0 <= a < AA:
                            term = f[t] * agg[a]
                            acc = term if acc is None else acc + term
                    ov[p - p0, l, pl.ds(off, LANES)] = acc
            return carry

        lax.fori_loop(0, NCH, chunk, 0)

    def do_block(i, xa, xp, first):
        for g, (ov, sem) in enumerate(((ovA, soA), (ovB, soB),
                                       (ovA, soA), (ovB, soB))):
            if g >= 2:
                wait_out(i, g - 2, ov, sem)
            elif first:
                @pl.when(i > 0)
                def _(g=g, ov=ov, sem=sem):
                    wait_out(i - 1, g + 2, ov, sem)
            else:
                wait_out(i - 1, g + 2, ov, sem)
            group_compute(g, ov, xa, xp)
            start_out(i, g, ov, sem)

    start_in(0, xa0, xp0, si0)

    def pair(j, carry):
        e = 2 * j
        o = e + 1
        start_in(o, xa1, xp1, si1)
        wait_in(e, xa0, xp0, si0)
        do_block(e, xa0, xp0, first=True)

        @pl.when(j + 1 < npair)
        def _():
            start_in(e + 2, xa0, xp0, si0)

        wait_in(o, xa1, xp1, si1)
        do_block(o, xa1, xp1, first=False)
        return carry

    lax.fori_loop(0, npair, pair, 0)
    wait_out(nblk - 1, 2, ovA, soA)
    wait_out(nblk - 1, 3, ovB, soB)


def kernel(x, W):
    B = x.shape[0]
    nblk = B // (NWORK * BLK)
    mesh = plsc.VectorSubcoreMesh(core_axis_name="c", subcore_axis_name="s")
    run = pl.kernel(
        lambda xh, wh, oh, wv, xa0, xa1, xp0, xp1, ovA, ovB, si0, si1, soA,
        soB: _body(xh, wh, oh, wv, xa0, xa1, xp0, xp1, ovA, ovB,
                   si0, si1, soA, soB, nblk),
        out_type=jax.ShapeDtypeStruct((NP, LOUT, B), jnp.float32),
        mesh=mesh,
        scratch_types=[
            pltpu.VMEM((AA * F,), jnp.float32),
            pltpu.VMEM((XA_ROWS, BLK), jnp.float32),
            pltpu.VMEM((XA_ROWS, BLK), jnp.float32),
            pltpu.VMEM((XP_ROWS, BLK), jnp.float32),
            pltpu.VMEM((XP_ROWS, BLK), jnp.float32),
            pltpu.VMEM((GMAX, LOUT + 2, BLK), jnp.float32),
            pltpu.VMEM((GMAX, LOUT + 2, BLK), jnp.float32),
            pltpu.SemaphoreType.DMA,
            pltpu.SemaphoreType.DMA,
            pltpu.SemaphoreType.DMA,
            pltpu.SemaphoreType.DMA,
        ],
        compiler_params=pltpu.CompilerParams(needs_layout_passes=False,
                                             use_tc_tiling_on_sc=True),
    )
    out = run(x.T, W.reshape(-1))
    return jnp.transpose(out, (2, 0, 1))


# speedup vs baseline: 1.1293x; 1.0020x over previous
import jax
import jax.numpy as jnp
from jax import lax
from jax.experimental import pallas as pl
from jax.experimental.pallas import tpu as pltpu
from jax.experimental.pallas import tpu_sc as plsc

AA = 20
MP = 15
F = 3
NP = 34
LOUT = AA + F - 1
LANES = 16
NWORK = 32
BLK = 128
NCH = BLK // LANES
XA_ROWS = 184
XP_OFF = 296
XP_ROWS = 39
GROUPS = ((0, 9), (9, 9), (18, 8), (26, 8))
GMAX = 9

_P2J = {0: [0], 1: [1, 2], 2: [0, 1], 3: [2], 4: [1], 6: [2, 3], 7: [3],
        10: [4], 12: [5], 14: [6, 7], 15: [7], 17: [8], 18: [5, 6], 19: [7],
        21: [8], 22: [7, 8], 24: [8], 25: [6], 27: [4], 28: [3], 30: [2],
        31: [1], 33: [0]}


def _body(xh, wh, oh, wv, xa0, xa1, xp0, xp1, ovA, ovB,
          si0, si1, soA, soB, nblk):
    cid = lax.axis_index("c")
    sid = lax.axis_index("s")
    wid = sid * 2 + cid
    npair = nblk // 2

    pltpu.sync_copy(wh, wv)

    zero = jnp.zeros((LANES,), jnp.float32)

    def hbase(i):
        return (wid * nblk + i) * BLK

    def in_pairs(i, xa, xp):
        return ((xh.at[pl.ds(0, XA_ROWS), pl.ds(hbase(i), BLK)], xa),
                (xh.at[pl.ds(XP_OFF, XP_ROWS), pl.ds(hbase(i), BLK)], xp))

    def start_in(i, xa, xp, sem):
        for src, dst in in_pairs(i, xa, xp):
            pltpu.async_copy(src, dst, sem)

    def wait_in(i, xa, xp, sem):
        for src, dst in in_pairs(i, xa, xp):
            pltpu.make_async_copy(src, dst, sem).wait()

    def out_pair(i, g, ov):
        p0, n = GROUPS[g]
        return (ov.at[pl.ds(0, n), pl.ds(0, LOUT), :],
                oh.at[pl.ds(p0, n), :, pl.ds(hbase(i), BLK)])

    def start_out(i, g, ov, sem):
        src, dst = out_pair(i, g, ov)
        pltpu.async_copy(src, dst, sem)

    def wait_out(i, g, ov, sem):
        src, dst = out_pair(i, g, ov)
        pltpu.make_async_copy(src, dst, sem).wait()

    def group_compute(g, ov, xa, xp):
        p0, n = GROUPS[g]

        def chunk(c, carry):
            off = c * LANES
            pep = {}

            def pvec(j, a):
                if (j, a) not in pep:
                    pep[(j, a)] = xa[1 + j * AA + a, pl.ds(off, LANES)]
                return pep[(j, a)]

            aggs = {}
            for p in range(p0, p0 + n):
                js = _P2J.get(p)
                if js is None or tuple(js) in aggs:
                    continue
                if len(js) == 1:
                    aggs[tuple(js)] = [pvec(js[0], a) for a in range(AA)]
                else:
                    aggs[tuple(js)] = [pvec(js[0], a) + pvec(js[1], a)
                                       for a in range(AA)]
            for p in range(p0, p0 + n):
                js = _P2J.get(p)
                if js is None:
                    for l in range(LOUT):
                        ov[p - p0, l, pl.ds(off, LANES)] = zero
                    continue
                agg = aggs[tuple(js)]
                fb = (xp[301 - XP_OFF + p, pl.ds(off, LANES)]
                      .astype(jnp.int32) * F)
                f = [plsc.load_gather(wv, [fb + t]) for t in range(F)]
                for l in range(LOUT):
                    acc = None
                    for t in range(F):
                        a = l - t
                        if 0 <= a < AA:
                            term = f[t] * agg[a]
                            acc = term if acc is None else acc + term
                    ov[p - p0, l, pl.ds(off, LANES)] = acc
            return carry

        lax.fori_loop(0, NCH, chunk, 0)

    def do_block(i, xa, xp, first):
        for g, (ov, sem) in enumerate(((ovA, soA), (ovB, soB),
                                       (ovA, soA), (ovB, soB))):
            if g >= 2:
                wait_out(i, g - 2, ov, sem)
            elif first:
                @pl.when(i > 0)
                def _(g=g, ov=ov, sem=sem):
                    wait_out(i - 1, g + 2, ov, sem)
            else:
                wait_out(i - 1, g + 2, ov, sem)
            group_compute(g, ov, xa, xp)
            start_out(i, g, ov, sem)

    start_in(0, xa0, xp0, si0)

    def pair(j, carry):
        e = 2 * j
        o = e + 1
        start_in(o, xa1, xp1, si1)
        wait_in(e, xa0, xp0, si0)
        do_block(e, xa0, xp0, first=True)

        @pl.when(j + 1 < npair)
        def _():
            start_in(e + 2, xa0, xp0, si0)

        wait_in(o, xa1, xp1, si1)
        do_block(o, xa1, xp1, first=False)
        return carry

    lax.fori_loop(0, npair, pair, 0)
    wait_out(nblk - 1, 2, ovA, soA)
    wait_out(nblk - 1, 3, ovB, soB)


def kernel(x, W):
    B = x.shape[0]
    nblk = B // (NWORK * BLK)
    mesh = plsc.VectorSubcoreMesh(core_axis_name="c", subcore_axis_name="s")
    run = pl.kernel(
        lambda xh, wh, oh, wv, xa0, xa1, xp0, xp1, ovA, ovB, si0, si1, soA,
        soB: _body(xh, wh, oh, wv, xa0, xa1, xp0, xp1, ovA, ovB,
                   si0, si1, soA, soB, nblk),
        out_type=jax.ShapeDtypeStruct((NP, LOUT, B), jnp.float32),
        mesh=mesh,
        scratch_types=[
            pltpu.VMEM((AA * F,), jnp.float32),
            pltpu.VMEM((XA_ROWS, BLK), jnp.float32),
            pltpu.VMEM((XA_ROWS, BLK), jnp.float32),
            pltpu.VMEM((XP_ROWS, BLK), jnp.float32),
            pltpu.VMEM((XP_ROWS, BLK), jnp.float32),
            pltpu.VMEM((GMAX, LOUT + 2, BLK), jnp.float32),
            pltpu.VMEM((GMAX, LOUT + 2, BLK), jnp.float32),
            pltpu.SemaphoreType.DMA,
            pltpu.SemaphoreType.DMA,
            pltpu.SemaphoreType.DMA,
            pltpu.SemaphoreType.DMA,
        ],
        compiler_params=pltpu.CompilerParams(needs_layout_passes=False,
                                             use_tc_tiling_on_sc=True),
    )
    out = run(x.T, W.reshape(-1))
    return jnp.transpose(out, (2, 0, 1))
